# baseline (speedup 1.0000x reference)
import functools

import jax
import jax.numpy as jnp
from jax import lax
from jax.experimental import pallas as pl
from jax.experimental.pallas import tpu as pltpu
from jax.experimental.pallas import tpu_sc as plsc

def _asf32(x):
    return plsc.bitcast(x, jnp.float32)


def _lane_perm(x, idx):
    dn = lax.GatherDimensionNumbers(
        offset_dims=(), collapsed_slice_dims=(0,), start_index_map=(0,))
    return lax.gather(x, idx[:, None], dn, slice_sizes=(1,),
                      mode=lax.GatherScatterMode.PROMISE_IN_BOUNDS)


N_NODES = 10000
N_EDGES = 320000
D_FEAT = 128

NW = 32
EPW = N_EDGES // NW
B = 80
NBUF = 5
NBLK = EPW // B
NGRP = NBLK // NBUF
LANES = 16
CHUNKS = B // LANES
DSLICES = D_FEAT // LANES
SUBE = 4
B2 = 2 * B
NSTRM = 4
BQ = B2 // NSTRM


def _body(h_ref, src_ref, dst_ref, out_ref,
          idx, rows,
          scores0, scores1, scores2, scores3, scores4,
          gsem0, gsem1, gsem2, gsem3, gsem4,
          ssem0, ssem1, ssem2, ssem3, ssem4,
          isem0, isem1, isem2, isem3, isem4):
    scores = [scores0, scores1, scores2, scores3, scores4]
    gsem = [gsem0, gsem1, gsem2, gsem3, gsem4]
    ssem = [ssem0, ssem1, ssem2, ssem3, ssem4]
    isem = [isem0, isem1, isem2, isem3, isem4]

    wid = lax.axis_index("s") * 2 + lax.axis_index("c")
    ebase = wid * EPW

    def _fire_idx(blk, slot):
        pltpu.make_async_copy(
            src_ref.at[pl.ds(ebase + blk * B, B)],
            idx.at[pl.ds(slot * B2, B)], isem[slot]).start()
        pltpu.make_async_copy(
            dst_ref.at[pl.ds(ebase + blk * B, B)],
            idx.at[pl.ds(slot * B2 + B, B)], isem[slot]).start()

    def _wait_idx(slot):
        pltpu.make_async_copy(
            src_ref.at[pl.ds(0, B)], idx.at[pl.ds(slot * B2, B)],
            isem[slot]).wait()
        pltpu.make_async_copy(
            src_ref.at[pl.ds(0, B)], idx.at[pl.ds(slot * B2 + B, B)],
            isem[slot]).wait()

    def _fire_gathers(slot):
        for q in range(NSTRM):
            pltpu.make_async_copy(
                h_ref.at[idx.at[pl.ds(slot * B2 + q * BQ, BQ)]],
                rows.at[slot, pl.ds(q * BQ, BQ)], gsem[slot]).start()

    def _wait_gathers(slot):
        for q in range(NSTRM):
            pltpu.make_async_copy(
                h_ref.at[idx.at[pl.ds(slot * B2 + q * BQ, BQ)]],
                rows.at[slot, pl.ds(q * BQ, BQ)], gsem[slot]).wait()

    def _wait_store(slot):
        pltpu.make_async_copy(
            scores[slot], out_ref.at[pl.ds(0, B)], ssem[slot]).wait()

    for b in range(NBUF):
        _fire_idx(b, b)
    for b in range(NBUF):
        _wait_idx(b)
        _fire_gathers(b)

    lane = lax.iota(jnp.int32, LANES)
    perms = [lane ^ sh for sh in (8, 4, 2, 1)]

    def group_body(g, carry):
        for b in range(NBUF):
            blk = g * NBUF + b
            _wait_gathers(b)

            @pl.when(g < NGRP - 1)
            def _():
                _fire_idx(blk + NBUF, b)

            @pl.when(g > 0)
            def _():
                _wait_store(b)

            rb = rows.at[b]
            scores_b = scores[b]

            def chunk_body(c, carry2):
                def quad_body(q, sv):
                    for jj in range(SUBE):
                        j = q * SUBE + jj
                        row = c * LANES + j
                        prods = [rb[row, pl.ds(k * LANES, LANES)] *
                                 rb[B + row, pl.ds(k * LANES, LANES)]
                                 for k in range(DSLICES)]
                        while len(prods) > 1:
                            prods = [prods[i] + prods[i + 1]
                                     for i in range(0, len(prods), 2)]
                        acc = prods[0]
                        for p in perms:
                            acc = acc + _lane_perm(acc, p)
                        sv = jnp.where(lane == j, acc, sv)
                    return sv

                score_vec = lax.fori_loop(
                    0, LANES // SUBE, quad_body,
                    jnp.zeros((LANES,), jnp.float32))
                scores_b[pl.ds(c * LANES, LANES)] = score_vec
                return carry2

            lax.fori_loop(0, CHUNKS, chunk_body, 0, unroll=False)

            pltpu.make_async_copy(
                scores[b], out_ref.at[pl.ds(ebase + blk * B, B)],
                ssem[b]).start()

            @pl.when(g < NGRP - 1)
            def _():
                _wait_idx(b)
                _fire_gathers(b)
        return carry

    lax.fori_loop(0, NGRP, group_body, 0, unroll=False)

    for b in range(NBUF):
        _wait_store(b)


@functools.partial(jax.jit, static_argnums=())
def _score_sc(h, src, dst):
    mesh = plsc.VectorSubcoreMesh(core_axis_name="c", subcore_axis_name="s")
    kfn = functools.partial(
        pl.kernel,
        out_type=jax.ShapeDtypeStruct((N_EDGES,), jnp.float32),
        mesh=mesh,
        scratch_types=[
            pltpu.VMEM((NBUF * B2,), jnp.int32),
            pltpu.VMEM((NBUF, B2, D_FEAT), jnp.float32),
        ] + [pltpu.VMEM((B,), jnp.float32)] * NBUF
          + [pltpu.SemaphoreType.DMA] * 15,
    )(_body)
    return kfn(h, src, dst)


def kernel(h, edge_index):
    src = edge_index[0].astype(jnp.int32)
    dst = edge_index[1].astype(jnp.int32)
    return _score_sc(h, src, dst)


# speedup vs baseline: 1.0049x; 1.0049x over previous
import functools

import jax
import jax.numpy as jnp
from jax import lax
from jax.experimental import pallas as pl
from jax.experimental.pallas import tpu as pltpu
from jax.experimental.pallas import tpu_sc as plsc

def _asf32(x):
    return plsc.bitcast(x, jnp.float32)


def _lane_perm(x, idx):
    dn = lax.GatherDimensionNumbers(
        offset_dims=(), collapsed_slice_dims=(0,), start_index_map=(0,))
    return lax.gather(x, idx[:, None], dn, slice_sizes=(1,),
                      mode=lax.GatherScatterMode.PROMISE_IN_BOUNDS)


N_NODES = 10000
N_EDGES = 320000
D_FEAT = 128

NW = 32
EPW = N_EDGES // NW
B = 80
NBUF = 5
NBLK = EPW // B
NGRP = NBLK // NBUF
LANES = 16
CHUNKS = B // LANES
DSLICES = D_FEAT // LANES
SUBE = 4
B2 = 2 * B
NSTRM = 2
BQ = B2 // NSTRM


def _body(h_ref, src_ref, dst_ref, out_ref,
          idx, rows,
          scores0, scores1, scores2, scores3, scores4,
          gsem0, gsem1, gsem2, gsem3, gsem4,
          ssem0, ssem1, ssem2, ssem3, ssem4,
          isem0, isem1, isem2, isem3, isem4):
    scores = [scores0, scores1, scores2, scores3, scores4]
    gsem = [gsem0, gsem1, gsem2, gsem3, gsem4]
    ssem = [ssem0, ssem1, ssem2, ssem3, ssem4]
    isem = [isem0, isem1, isem2, isem3, isem4]

    wid = lax.axis_index("s") * 2 + lax.axis_index("c")
    ebase = wid * EPW

    def _fire_idx(blk, slot):
        pltpu.make_async_copy(
            src_ref.at[pl.ds(ebase + blk * B, B)],
            idx.at[pl.ds(slot * B2, B)], isem[slot]).start()
        pltpu.make_async_copy(
            dst_ref.at[pl.ds(ebase + blk * B, B)],
            idx.at[pl.ds(slot * B2 + B, B)], isem[slot]).start()

    def _wait_idx(slot):
        pltpu.make_async_copy(
            src_ref.at[pl.ds(0, B)], idx.at[pl.ds(slot * B2, B)],
            isem[slot]).wait()
        pltpu.make_async_copy(
            src_ref.at[pl.ds(0, B)], idx.at[pl.ds(slot * B2 + B, B)],
            isem[slot]).wait()

    def _fire_gathers(slot):
        for q in range(NSTRM):
            pltpu.make_async_copy(
                h_ref.at[idx.at[pl.ds(slot * B2 + q * BQ, BQ)]],
                rows.at[slot, pl.ds(q * BQ, BQ)], gsem[slot]).start()

    def _wait_gathers(slot):
        for q in range(NSTRM):
            pltpu.make_async_copy(
                h_ref.at[idx.at[pl.ds(slot * B2 + q * BQ, BQ)]],
                rows.at[slot, pl.ds(q * BQ, BQ)], gsem[slot]).wait()

    def _wait_store(slot):
        pltpu.make_async_copy(
            scores[slot], out_ref.at[pl.ds(0, B)], ssem[slot]).wait()

    for b in range(NBUF):
        _fire_idx(b, b)
    for b in range(NBUF):
        _wait_idx(b)
        _fire_gathers(b)

    lane = lax.iota(jnp.int32, LANES)
    perms = [lane ^ sh for sh in (8, 4, 2, 1)]

    def group_body(g, carry):
        for b in range(NBUF):
            blk = g * NBUF + b
            _wait_gathers(b)

            @pl.when(g < NGRP - 1)
            def _():
                _fire_idx(blk + NBUF, b)

            @pl.when(g > 0)
            def _():
                _wait_store(b)

            rb = rows.at[b]
            scores_b = scores[b]

            def chunk_body(c, carry2):
                def quad_body(q, sv):
                    for jj in range(SUBE):
                        j = q * SUBE + jj
                        row = c * LANES + j
                        prods = [rb[row, pl.ds(k * LANES, LANES)] *
                                 rb[B + row, pl.ds(k * LANES, LANES)]
                                 for k in range(DSLICES)]
                        while len(prods) > 1:
                            prods = [prods[i] + prods[i + 1]
                                     for i in range(0, len(prods), 2)]
                        acc = prods[0]
                        for p in perms:
                            acc = acc + _lane_perm(acc, p)
                        sv = jnp.where(lane == j, acc, sv)
                    return sv

                score_vec = lax.fori_loop(
                    0, LANES // SUBE, quad_body,
                    jnp.zeros((LANES,), jnp.float32))
                scores_b[pl.ds(c * LANES, LANES)] = score_vec
                return carry2

            lax.fori_loop(0, CHUNKS, chunk_body, 0, unroll=False)

            pltpu.make_async_copy(
                scores[b], out_ref.at[pl.ds(ebase + blk * B, B)],
                ssem[b]).start()

            @pl.when(g < NGRP - 1)
            def _():
                _wait_idx(b)
                _fire_gathers(b)
        return carry

    lax.fori_loop(0, NGRP, group_body, 0, unroll=False)

    for b in range(NBUF):
        _wait_store(b)


@functools.partial(jax.jit, static_argnums=())
def _score_sc(h, src, dst):
    mesh = plsc.VectorSubcoreMesh(core_axis_name="c", subcore_axis_name="s")
    kfn = functools.partial(
        pl.kernel,
        out_type=jax.ShapeDtypeStruct((N_EDGES,), jnp.float32),
        mesh=mesh,
        scratch_types=[
            pltpu.VMEM((NBUF * B2,), jnp.int32),
            pltpu.VMEM((NBUF, B2, D_FEAT), jnp.float32),
        ] + [pltpu.VMEM((B,), jnp.float32)] * NBUF
          + [pltpu.SemaphoreType.DMA] * 15,
    )(_body)
    return kfn(h, src, dst)


def kernel(h, edge_index):
    src = edge_index[0].astype(jnp.int32)
    dst = edge_index[1].astype(jnp.int32)
    return _score_sc(h, src, dst)


# speedup vs baseline: 1.0345x; 1.0294x over previous
import functools

import jax
import jax.numpy as jnp
from jax import lax
from jax.experimental import pallas as pl
from jax.experimental.pallas import tpu as pltpu
from jax.experimental.pallas import tpu_sc as plsc

def _lane_perm(x, idx):
    dn = lax.GatherDimensionNumbers(
        offset_dims=(), collapsed_slice_dims=(0,), start_index_map=(0,))
    return lax.gather(x, idx[:, None], dn, slice_sizes=(1,),
                      mode=lax.GatherScatterMode.PROMISE_IN_BOUNDS)


N_NODES = 10000
N_EDGES = 320000
D_FEAT = 128

NW = 32
EPW = N_EDGES // NW
B = 80
NBUF = 5
NBLK = EPW // B
NGRP = NBLK // NBUF
LANES = 16
CHUNKS = B // LANES
DSLICES = D_FEAT // LANES
SUBE = 4


def _body(h_ref, src_ref, dst_ref, out_ref,
          idx_u, idx_v, rows_u, rows_v,
          scores0, scores1, scores2, scores3, scores4,
          gsem0, gsem1, gsem2, gsem3, gsem4,
          ssem0, ssem1, ssem2, ssem3, ssem4,
          isem0, isem1, isem2, isem3, isem4):
    scores = [scores0, scores1, scores2, scores3, scores4]
    gsem = [gsem0, gsem1, gsem2, gsem3, gsem4]
    ssem = [ssem0, ssem1, ssem2, ssem3, ssem4]
    isem = [isem0, isem1, isem2, isem3, isem4]

    wid = lax.axis_index("s") * 2 + lax.axis_index("c")
    ebase = wid * EPW

    def _fire_idx(blk, slot):
        pltpu.make_async_copy(
            src_ref.at[pl.ds(ebase + blk * B, B)], idx_u.at[slot],
            isem[slot]).start()
        pltpu.make_async_copy(
            dst_ref.at[pl.ds(ebase + blk * B, B)], idx_v.at[slot],
            isem[slot]).start()

    def _wait_idx(slot):
        pltpu.make_async_copy(
            src_ref.at[pl.ds(0, B)], idx_u.at[slot], isem[slot]).wait()
        pltpu.make_async_copy(
            dst_ref.at[pl.ds(0, B)], idx_v.at[slot], isem[slot]).wait()

    def _fire_gathers(slot):
        pltpu.make_async_copy(
            h_ref.at[idx_u.at[slot]], rows_u.at[slot], gsem[slot]).start()
        pltpu.make_async_copy(
            h_ref.at[idx_v.at[slot]], rows_v.at[slot], gsem[slot]).start()

    def _wait_gathers(slot):
        pltpu.make_async_copy(
            h_ref.at[idx_u.at[slot]], rows_u.at[slot], gsem[slot]).wait()
        pltpu.make_async_copy(
            h_ref.at[idx_v.at[slot]], rows_v.at[slot], gsem[slot]).wait()

    def _wait_store(slot):
        pltpu.make_async_copy(
            scores[slot], out_ref.at[pl.ds(0, B)], ssem[slot]).wait()

    for b in range(NBUF):
        _fire_idx(b, b)
    for b in range(NBUF):
        _wait_idx(b)
        _fire_gathers(b)

    lane = lax.iota(jnp.int32, LANES)
    perms = [lane ^ sh for sh in (8, 4, 2, 1)]

    def group_body(g, carry):
        for b in range(NBUF):
            blk = g * NBUF + b
            _wait_gathers(b)

            @pl.when(g < NGRP - 1)
            def _():
                _fire_idx(blk + NBUF, b)

            @pl.when(g > 0)
            def _():
                _wait_store(b)

            ru = rows_u.at[b]
            rv = rows_v.at[b]
            scores_b = scores[b]

            def chunk_body(c, carry2):
                def quad_body(q, sv):
                    for jj in range(SUBE):
                        j = q * SUBE + jj
                        row = c * LANES + j
                        prods = [ru[row, pl.ds(k * LANES, LANES)] *
                                 rv[row, pl.ds(k * LANES, LANES)]
                                 for k in range(DSLICES)]
                        while len(prods) > 1:
                            prods = [prods[i] + prods[i + 1]
                                     for i in range(0, len(prods), 2)]
                        acc = prods[0]
                        for p in perms:
                            acc = acc + _lane_perm(acc, p)
                        sv = jnp.where(lane == j, acc, sv)
                    return sv

                score_vec = lax.fori_loop(
                    0, LANES // SUBE, quad_body,
                    jnp.zeros((LANES,), jnp.float32))
                scores_b[pl.ds(c * LANES, LANES)] = score_vec
                return carry2

            lax.fori_loop(0, CHUNKS, chunk_body, 0, unroll=False)

            pltpu.make_async_copy(
                scores[b], out_ref.at[pl.ds(ebase + blk * B, B)],
                ssem[b]).start()

            @pl.when(g < NGRP - 1)
            def _():
                _wait_idx(b)
                _fire_gathers(b)
        return carry

    lax.fori_loop(0, NGRP, group_body, 0, unroll=False)

    for b in range(NBUF):
        _wait_store(b)


@functools.partial(jax.jit, static_argnums=())
def _score_sc(h, src, dst):
    mesh = plsc.VectorSubcoreMesh(core_axis_name="c", subcore_axis_name="s")
    kfn = functools.partial(
        pl.kernel,
        out_type=jax.ShapeDtypeStruct((N_EDGES,), jnp.float32),
        mesh=mesh,
        scratch_types=[
            pltpu.VMEM((NBUF, B), jnp.int32),
            pltpu.VMEM((NBUF, B), jnp.int32),
            pltpu.VMEM((NBUF, B, D_FEAT), jnp.float32),
            pltpu.VMEM((NBUF, B, D_FEAT), jnp.float32),
        ] + [pltpu.VMEM((B,), jnp.float32)] * NBUF
          + [pltpu.SemaphoreType.DMA] * 15,
    )(_body)
    return kfn(h, src, dst)


def kernel(h, edge_index):
    src = edge_index[0].astype(jnp.int32)
    dst = edge_index[1].astype(jnp.int32)
    return _score_sc(h, src, dst)
